# Initial kernel scaffold; baseline (speedup 1.0000x reference)
import jax
import jax.numpy as jnp
from jax.experimental import pallas as pl


def kernel(x, edge_index, batch, edge_attr, We1, W1, Ws1, b1, We2, W2, Ws2, b2, Wext1, bext1, Wext2, bext2, Wout, bout, Wm, bm):
    raise NotImplementedError("write your pallas kernel here")


# speedup vs baseline: 5.6193x; 5.6193x over previous
import functools

import jax
import jax.numpy as jnp
from jax import lax
from jax.experimental import pallas as pl
from jax.experimental.pallas import tpu as pltpu
from jax.experimental.pallas import tpu_sc as plsc

N_NODES = 10000
N_EDGES = 320000
D = 128
DE = 4
G = 64
EAP = 16

NC = 2
NS = 16
NW = NC * NS
EPW = N_EDGES // NW
K = 80
NCHUNK = EPW // K
ZR = 640
ZR_LAST = N_NODES - (NS - 1) * ZR

NB = 2000
NBLK = N_NODES // NB

_f32 = jnp.float32


def _sc_mesh():
  return plsc.VectorSubcoreMesh(
      core_axis_name="c", subcore_axis_name="s", num_cores=NC,
      num_subcores=NS)


def _make_sc_pass(with_ea: bool, with_att: bool):
  out_type = [jax.ShapeDtypeStruct((NC * N_NODES, D), _f32)]
  if with_ea:
    out_type.append(jax.ShapeDtypeStruct((NC * N_NODES, EAP), _f32))

  scratch = [
      pltpu.VMEM((K,), jnp.int32),
      pltpu.VMEM((K,), jnp.int32),
      pltpu.VMEM((K, D), _f32),
      pltpu.VMEM_SHARED((N_NODES, D), _f32),
      pltpu.SemaphoreType.DMA,
  ]
  if with_ea:
    scratch += [
        pltpu.VMEM((K, EAP), _f32),
        pltpu.VMEM_SHARED((N_NODES, EAP), _f32),
    ]
  if with_att:
    scratch += [
        pltpu.VMEM((K, EAP), _f32),
        pltpu.SemaphoreType.DMA,
    ]

  def body(*refs):
    it = iter(refs)
    y = next(it)
    src = next(it)
    dst = next(it)
    zeros = next(it)
    ea = next(it) if with_ea else None
    z16 = next(it) if with_ea else None
    att = next(it) if with_att else None
    s_out = next(it)
    ea_out = next(it) if with_ea else None
    src_i = next(it)
    dst_i = next(it)
    rows = next(it)
    acc = next(it)
    sem = next(it)
    ea_b = next(it) if with_ea else None
    acc_ea = next(it) if with_ea else None
    att_r = next(it) if with_att else None
    sem_a = next(it) if with_att else None

    c = lax.axis_index("c")
    s = lax.axis_index("s")
    wid = s * NC + c
    r0 = s * ZR

    def _zero(rows_n):
      pltpu.sync_copy(zeros.at[pl.ds(r0, rows_n)], acc.at[pl.ds(r0, rows_n)])
      if with_ea:
        pltpu.sync_copy(z16.at[pl.ds(r0, rows_n)],
                        acc_ea.at[pl.ds(r0, rows_n)])

    @pl.when(s < NS - 1)
    def _():
      _zero(ZR)

    @pl.when(s == NS - 1)
    def _():
      _zero(ZR_LAST)

    plsc.subcore_barrier()

    def chunk(i, carry):
      base = wid * EPW + i * K
      pltpu.sync_copy(src.at[pl.ds(base, K)], src_i)
      pltpu.sync_copy(dst.at[pl.ds(base, K)], dst_i)
      cp = pltpu.async_copy(y.at[src_i], rows, sem)
      if with_att:
        cp_a = pltpu.async_copy(att.at[src_i], att_r, sem_a)
      if with_ea:
        pltpu.sync_copy(ea.at[pl.ds(base, K)], ea_b)
      cp.wait()
      if with_att:
        cp_a.wait()

        def mul(j, carry2):
          ea_b[j, :] = ea_b[j, :] * att_r[j, :]
          return carry2

        lax.fori_loop(0, K, mul, 0)
      pltpu.sync_copy(rows, acc.at[dst_i], add=True)
      if with_ea:
        pltpu.sync_copy(ea_b, acc_ea.at[dst_i], add=True)
      return carry

    lax.fori_loop(0, NCHUNK, chunk, 0)
    plsc.subcore_barrier()

    o0 = c * N_NODES + r0

    def _wb(rows_n):
      pltpu.sync_copy(acc.at[pl.ds(r0, rows_n)], s_out.at[pl.ds(o0, rows_n)])
      if with_ea:
        pltpu.sync_copy(acc_ea.at[pl.ds(r0, rows_n)],
                        ea_out.at[pl.ds(o0, rows_n)])

    @pl.when(s < NS - 1)
    def _():
      _wb(ZR)

    @pl.when(s == NS - 1)
    def _():
      _wb(ZR_LAST)

  return pl.kernel(
      body, out_type=out_type, mesh=_sc_mesh(),
      scratch_types=scratch,
      compiler_params=pltpu.CompilerParams(use_tc_tiling_on_sc=False))


_sc_pass_plain = _make_sc_pass(with_ea=False, with_att=False)
_sc_pass_ea = _make_sc_pass(with_ea=True, with_att=False)
_sc_pass_ea_att = _make_sc_pass(with_ea=True, with_att=True)


def _dot(a, b):
  return jnp.dot(a, b, preferred_element_type=_f32)


def _part_specs():
  return [
      pl.BlockSpec((NB, D), lambda i: (i, 0)),
      pl.BlockSpec((NB, D), lambda i: (i + NBLK, 0)),
      pl.BlockSpec((NB, EAP), lambda i: (i, 0)),
      pl.BlockSpec((NB, EAP), lambda i: (i + NBLK, 0)),
  ]


def _w_spec(shape):
  return pl.BlockSpec(shape, lambda i: tuple(0 for _ in shape))


def _layer1_body(sa, sb, pa, pb, x, we, w, ws, b, o_h1):
  agg = sa[...] + sb[...] + _dot((pa[...] + pb[...])[:, :DE], we[...])
  o_h1[...] = jax.nn.relu(_dot(agg, w[...]) + _dot(x[...], ws[...]) + b[...])


def _layer2_body(sa, sb, pa, pb, h1, x, we, w, ws, b, we1, be1, we2, be2,
                 o_att, o_y3):
  agg = sa[...] + sb[...] + _dot((pa[...] + pb[...])[:, :DE], we[...])
  emb = jax.nn.relu(_dot(agg, w[...]) + _dot(h1[...], ws[...]) + b[...])
  logit = _dot(jax.nn.relu(_dot(emb, we1[...]) + be1[...]), we2[...]) + be2[...]
  att = jax.nn.sigmoid(logit)
  o_att[...] = jnp.broadcast_to(att, (NB, EAP))
  o_y3[...] = x[...] * att


def _layer3_body(sa, sb, pa, pb, attp, x, we, w, ws, b, o_g1, o_y4):
  att = attp[:, 0:1]
  agg = (sa[...] + sb[...] + _dot((pa[...] + pb[...])[:, :DE], we[...])) * att
  g1 = jax.nn.relu(_dot(agg, w[...]) + _dot(x[...], ws[...]) + b[...])
  o_g1[...] = g1
  o_y4[...] = g1 * att


def _layer4_body(sa, sb, pa, pb, attp, g1, we, w, ws, b, o_g2):
  att = attp[:, 0:1]
  agg = (sa[...] + sb[...] + _dot((pa[...] + pb[...])[:, :DE], we[...])) * att
  o_g2[...] = jax.nn.relu(_dot(agg, w[...]) + _dot(g1[...], ws[...]) + b[...])


def _pool_body(g2, bt, wout, bout, wm, bm, o, acc, cnt):
  i = pl.program_id(0)

  @pl.when(i == 0)
  def _():
    acc[...] = jnp.zeros_like(acc)
    cnt[...] = jnp.zeros_like(cnt)

  gids = lax.broadcasted_iota(jnp.int32, (G, NB), 0)
  mask = (bt[0, 0, :][None, :] == gids).astype(_f32)
  acc[...] += _dot(mask, g2[...])
  cnt[...] += jnp.broadcast_to(jnp.sum(mask, axis=1, keepdims=True), (G, D))

  @pl.when(i == NBLK - 1)
  def _():
    pooled = acc[...] / jnp.maximum(cnt[...], 1.0)
    logits = _dot(pooled, wout[...]) + bout[...]
    o[...] = _dot(logits, wm[...]) + bm[...]


def _tc_layer1(sp, pp, x, we, w, ws, b):
  specs = _part_specs() + [
      pl.BlockSpec((NB, D), lambda i: (i, 0)),
      _w_spec((DE, D)), _w_spec((D, D)), _w_spec((D, D)), _w_spec((1, D)),
  ]
  return pl.pallas_call(
      _layer1_body, grid=(NBLK,), in_specs=specs,
      out_specs=pl.BlockSpec((NB, D), lambda i: (i, 0)),
      out_shape=jax.ShapeDtypeStruct((N_NODES, D), _f32),
  )(sp, sp, pp, pp, x, we, w, ws, b)


def _tc_layer2(sp, pp, h1, x, we, w, ws, b, we1, be1, we2, be2):
  specs = _part_specs() + [
      pl.BlockSpec((NB, D), lambda i: (i, 0)),
      pl.BlockSpec((NB, D), lambda i: (i, 0)),
      _w_spec((DE, D)), _w_spec((D, D)), _w_spec((D, D)), _w_spec((1, D)),
      _w_spec((D, 64)), _w_spec((1, 64)), _w_spec((64, 1)), _w_spec((1, 1)),
  ]
  return pl.pallas_call(
      _layer2_body, grid=(NBLK,), in_specs=specs,
      out_specs=[pl.BlockSpec((NB, EAP), lambda i: (i, 0)),
                 pl.BlockSpec((NB, D), lambda i: (i, 0))],
      out_shape=[jax.ShapeDtypeStruct((N_NODES, EAP), _f32),
                 jax.ShapeDtypeStruct((N_NODES, D), _f32)],
  )(sp, sp, pp, pp, h1, x, we, w, ws, b, we1, be1, we2, be2)


def _tc_layer3(sp, pp, attp, x, we, w, ws, b):
  specs = _part_specs() + [
      pl.BlockSpec((NB, EAP), lambda i: (i, 0)),
      pl.BlockSpec((NB, D), lambda i: (i, 0)),
      _w_spec((DE, D)), _w_spec((D, D)), _w_spec((D, D)), _w_spec((1, D)),
  ]
  return pl.pallas_call(
      _layer3_body, grid=(NBLK,), in_specs=specs,
      out_specs=[pl.BlockSpec((NB, D), lambda i: (i, 0)),
                 pl.BlockSpec((NB, D), lambda i: (i, 0))],
      out_shape=[jax.ShapeDtypeStruct((N_NODES, D), _f32),
                 jax.ShapeDtypeStruct((N_NODES, D), _f32)],
  )(sp, sp, pp, pp, attp, x, we, w, ws, b)


def _tc_layer4(sp, pp, attp, g1, we, w, ws, b):
  specs = _part_specs() + [
      pl.BlockSpec((NB, EAP), lambda i: (i, 0)),
      pl.BlockSpec((NB, D), lambda i: (i, 0)),
      _w_spec((DE, D)), _w_spec((D, D)), _w_spec((D, D)), _w_spec((1, D)),
  ]
  return pl.pallas_call(
      _layer4_body, grid=(NBLK,), in_specs=specs,
      out_specs=pl.BlockSpec((NB, D), lambda i: (i, 0)),
      out_shape=jax.ShapeDtypeStruct((N_NODES, D), _f32),
  )(sp, sp, pp, pp, attp, g1, we, w, ws, b)


def _tc_pool(g2, batch_r, wout, bout, wm, bm):
  specs = [
      pl.BlockSpec((NB, D), lambda i: (i, 0)),
      pl.BlockSpec((1, 1, NB), lambda i: (i, 0, 0)),
      _w_spec((D, 1)), _w_spec((1, 1)), _w_spec((1, 2)), _w_spec((1, 2)),
  ]
  return pl.pallas_call(
      _pool_body, grid=(NBLK,), in_specs=specs,
      out_specs=pl.BlockSpec((G, 2), lambda i: (0, 0)),
      out_shape=jax.ShapeDtypeStruct((G, 2), _f32),
      scratch_shapes=[pltpu.VMEM((G, D), _f32), pltpu.VMEM((G, D), _f32)],
  )(g2, batch_r, wout, bout, wm, bm)


@jax.jit
def kernel(x, edge_index, batch, edge_attr, We1, W1, Ws1, b1, We2, W2, Ws2,
           b2, Wext1, bext1, Wext2, bext2, Wout, bout, Wm, bm):
  src = edge_index[0]
  dst = edge_index[1]
  ea_pad = jnp.zeros((N_EDGES, EAP), _f32).at[:, :DE].set(edge_attr)
  zeros = jnp.zeros((N_NODES, D), _f32)
  z16 = jnp.zeros((N_NODES, EAP), _f32)
  b1r = b1.reshape(1, D)
  b2r = b2.reshape(1, D)
  be1 = bext1.reshape(1, 64)
  be2 = bext2.reshape(1, 1)
  boutr = bout.reshape(1, 1)
  bmr = bm.reshape(1, 2)
  batch_r = batch.reshape(NBLK, 1, NB)

  s1, p4a = _sc_pass_ea(x, src, dst, zeros, ea_pad, z16)
  h1 = _tc_layer1(s1, p4a, x, We1, W1, Ws1, b1r)

  (s2,) = _sc_pass_plain(h1, src, dst, zeros)
  attp, y3 = _tc_layer2(s2, p4a, h1, x, We2, W2, Ws2, b2r,
                        Wext1, be1, Wext2, be2)

  s3, p4b = _sc_pass_ea_att(y3, src, dst, zeros, ea_pad, z16, attp)
  g1, y4 = _tc_layer3(s3, p4b, attp, x, We1, W1, Ws1, b1r)

  (s4,) = _sc_pass_plain(y4, src, dst, zeros)
  g2 = _tc_layer4(s4, p4b, attp, g1, We2, W2, Ws2, b2r)

  return _tc_pool(g2, batch_r, Wout, boutr, Wm, bmr)
